# baseline (speedup 1.0000x reference)
import functools

import jax
import jax.numpy as jnp
from jax.experimental import pallas as pl
from jax.experimental.pallas import tpu as pltpu

_ALPHA = 0.2
_NEG = -9e15


def _gat_kernel(adj_ref, h_ref, w_ref, b_ref, a1_ref, a2_ref, o_ref,
                wh_ref, d_ref, *, blk):
    i = pl.program_id(0)

    @pl.when(i == 0)
    def _prep():
        wh0 = jax.lax.dot_general(
            h_ref[...], w_ref[...], (((1,), (1,)), ((), ())),
            preferred_element_type=jnp.float32,
        ) + b_ref[...]
        wh_ref[...] = wh0
        d_ref[...] = jax.lax.dot_general(
            a2_ref[...], wh0, (((1,), (1,)), ((), ())),
            preferred_element_type=jnp.float32,
        )

    wh = wh_ref[...]
    wh_blk = wh_ref[pl.ds(i * blk, blk), :]
    s = jnp.sum(wh_blk * a1_ref[...], axis=1, keepdims=True)
    logits = s + d_ref[...]
    e = jnp.maximum(logits, _ALPHA * logits)
    e = jnp.where(adj_ref[...] != 0, e, _NEG)
    m = jnp.max(e, axis=1, keepdims=True)
    p = jnp.exp(e - m)
    acc = jnp.dot(p, wh, preferred_element_type=jnp.float32)
    o_ref[...] = acc / jnp.sum(p, axis=1, keepdims=True)


def kernel(h, adj_matrix, W_weight, W_bias, a):
    n, _ = h.shape
    out_dim = W_weight.shape[0]
    b2 = W_bias.reshape(1, out_dim)
    a1 = a[:, :out_dim]
    a2 = a[:, out_dim:]

    blk = 128
    grid = n // blk
    out = pl.pallas_call(
        functools.partial(_gat_kernel, blk=blk),
        grid=(grid,),
        in_specs=[
            pl.BlockSpec((blk, n), lambda i: (i, 0)),
            pl.BlockSpec(h.shape, lambda i: (0, 0)),
            pl.BlockSpec(W_weight.shape, lambda i: (0, 0)),
            pl.BlockSpec((1, out_dim), lambda i: (0, 0)),
            pl.BlockSpec((1, out_dim), lambda i: (0, 0)),
            pl.BlockSpec((1, out_dim), lambda i: (0, 0)),
        ],
        out_specs=pl.BlockSpec((blk, out_dim), lambda i: (i, 0)),
        out_shape=jax.ShapeDtypeStruct((n, out_dim), jnp.float32),
        scratch_shapes=[
            pltpu.VMEM((n, out_dim), jnp.float32),
            pltpu.VMEM((1, n), jnp.float32),
        ],
        compiler_params=pltpu.CompilerParams(
            dimension_semantics=("arbitrary",),
        ),
    )(adj_matrix, h, W_weight, b2, a1, a2)
    return out


# speedup vs baseline: 1.3762x; 1.3762x over previous
import functools

import jax
import jax.numpy as jnp
from jax.experimental import pallas as pl
from jax.experimental.pallas import tpu as pltpu

_ALPHA = 0.2
_NEG = -9e15


def _gat_kernel(adj_ref, h_ref, w_ref, b_ref, a1_ref, a2_ref, o_ref,
                wh_ref, d_ref, *, blk):
    i = pl.program_id(0)

    @pl.when(i == 0)
    def _prep():
        wh0 = jax.lax.dot_general(
            h_ref[...], w_ref[...], (((1,), (1,)), ((), ())),
            preferred_element_type=jnp.float32,
        ) + b_ref[...]
        wh_ref[...] = wh0
        d_ref[...] = jax.lax.dot_general(
            a2_ref[...], wh0, (((1,), (1,)), ((), ())),
            preferred_element_type=jnp.float32,
        )

    wh = wh_ref[...]
    wh_blk = wh_ref[pl.ds(i * blk, blk), :]
    s = jnp.sum(wh_blk * a1_ref[...], axis=1, keepdims=True)
    logits = s + d_ref[...]
    e = jnp.maximum(logits, _ALPHA * logits)
    e = jnp.where(adj_ref[...] != 0, e, _NEG)
    m = jnp.max(e, axis=1, keepdims=True)
    p = jnp.exp(e - m)
    acc = jnp.dot(p, wh, preferred_element_type=jnp.float32)
    o_ref[...] = acc / jnp.sum(p, axis=1, keepdims=True)


def kernel(h, adj_matrix, W_weight, W_bias, a):
    n, _ = h.shape
    out_dim = W_weight.shape[0]
    b2 = W_bias.reshape(1, out_dim)
    a1 = a[:, :out_dim]
    a2 = a[:, out_dim:]

    blk = 512
    grid = n // blk
    out = pl.pallas_call(
        functools.partial(_gat_kernel, blk=blk),
        grid=(grid,),
        in_specs=[
            pl.BlockSpec((blk, n), lambda i: (i, 0)),
            pl.BlockSpec(h.shape, lambda i: (0, 0)),
            pl.BlockSpec(W_weight.shape, lambda i: (0, 0)),
            pl.BlockSpec((1, out_dim), lambda i: (0, 0)),
            pl.BlockSpec((1, out_dim), lambda i: (0, 0)),
            pl.BlockSpec((1, out_dim), lambda i: (0, 0)),
        ],
        out_specs=pl.BlockSpec((blk, out_dim), lambda i: (i, 0)),
        out_shape=jax.ShapeDtypeStruct((n, out_dim), jnp.float32),
        scratch_shapes=[
            pltpu.VMEM((n, out_dim), jnp.float32),
            pltpu.VMEM((1, n), jnp.float32),
        ],
        compiler_params=pltpu.CompilerParams(
            dimension_semantics=("arbitrary",),
        ),
    )(adj_matrix, h, W_weight, b2, a1, a2)
    return out
